# baseline (speedup 1.0000x reference)
import functools
import math

import jax
import jax.numpy as jnp
from jax import lax
from jax.experimental import pallas as pl
from jax.experimental.pallas import tpu as pltpu
from jax.experimental.pallas import tpu_sc as plsc

D_MODEL = 64
SCALE = math.sqrt(D_MODEL)
NC, NS = 2, 16
NW = NC * NS
CHUNK = 512
NBUF = 2
LANES = 16


@functools.partial(jax.jit, static_argnames=("total",))
def _emb_lookup(x_flat, table, *, total):
    b_per_w = total // NW
    nchunks = b_per_w // CHUNK
    assert nchunks % NBUF == 0 and nchunks // NBUF >= 2

    mesh = plsc.VectorSubcoreMesh(core_axis_name="c", subcore_axis_name="s")

    @functools.partial(
        pl.kernel,
        out_type=jax.ShapeDtypeStruct((total, D_MODEL), jnp.float32),
        mesh=mesh,
        scratch_types=(
            [pltpu.VMEM((CHUNK,), jnp.int32) for _ in range(NBUF)]
            + [pltpu.VMEM((CHUNK, D_MODEL), jnp.float32) for _ in range(NBUF)]
            + [pltpu.SemaphoreType.DMA for _ in range(2 * NBUF)]
        ),
        compiler_params=pltpu.CompilerParams(use_tc_tiling_on_sc=False),
    )
    def body(x_hbm, table_hbm, out_hbm, *scratch):
        idx = scratch[:NBUF]
        rows = scratch[NBUF:2 * NBUF]
        gsem = scratch[2 * NBUF:3 * NBUF]
        osem = scratch[3 * NBUF:]

        wid = lax.axis_index("s") * NC + lax.axis_index("c")
        base = pl.multiple_of(wid * b_per_w, CHUNK)

        def start_gather(b, g):
            start = pl.multiple_of(base + g * CHUNK, CHUNK)
            pltpu.sync_copy(x_hbm.at[pl.ds(start, CHUNK)], idx[b])
            pltpu.async_copy(table_hbm.at[idx[b]], rows[b], gsem[b])

        def wait_gather(b):
            pltpu.make_async_copy(table_hbm.at[idx[b]], rows[b], gsem[b]).wait()

        def scale_buf(b):
            def row_body(r, carry):
                for j in range(D_MODEL // LANES):
                    sl = pl.ds(j * LANES, LANES)
                    rows[b][r, sl] = rows[b][r, sl] * SCALE
                return carry
            lax.fori_loop(0, CHUNK, row_body, 0, unroll=2)

        def start_write(b, g):
            start = pl.multiple_of(base + g * CHUNK, CHUNK)
            pltpu.async_copy(rows[b], out_hbm.at[pl.ds(start, CHUNK)], osem[b])

        def wait_write(b, g):
            start = pl.multiple_of(base + g * CHUNK, CHUNK)
            pltpu.make_async_copy(
                rows[b], out_hbm.at[pl.ds(start, CHUNK)], osem[b]
            ).wait()

        for b in range(NBUF):
            start_gather(b, b)

        def main(i, carry):
            for b in range(NBUF):
                g = i * NBUF + b
                wait_gather(b)
                scale_buf(b)
                start_write(b, g)
                wait_write(b, g)
                start_gather(b, g + NBUF)
            return carry

        lax.fori_loop(0, nchunks // NBUF - 1, main, 0)

        for b in range(NBUF):
            g = nchunks - NBUF + b
            wait_gather(b)
            scale_buf(b)
            start_write(b, g)
            wait_write(b, g)

    return body(x_flat, table)


def kernel(x, table):
    total = x.shape[0] * x.shape[1]
    out = _emb_lookup(x.reshape(total), table, total=total)
    return out.reshape(x.shape[0], x.shape[1], D_MODEL)
